# baseline (speedup 1.0000x reference)
import functools

import jax
import jax.numpy as jnp
from jax import lax
from jax.experimental import pallas as pl
from jax.experimental.pallas import tpu as pltpu
from jax.experimental.pallas import tpu_sc as plsc

N = 10000
E = 320000
M = 640000
NP = 10240
NH = 10112
F1 = 128
F1P = 144
F2 = 64
NWK = 32
CH = 128
CW = 160
CW0 = 160
CW1 = 160
MP = NWK * CW * CH

_mesh = functools.partial(
    plsc.VectorSubcoreMesh, core_axis_name="c", subcore_axis_name="s")


def _f32(*shape):
    return jax.ShapeDtypeStruct(shape, jnp.float32)


def _prep_body(nf, nwt, ei0, ei1, he1, fu, fv, vr, idx_v, rows_v, sem):
    cid = lax.axis_index("c")
    sid = lax.axis_index("s")
    wid = sid * 2 + cid
    for idx_hbm, tab, out in ((ei0, nf, fu), (ei1, nf, fv), (he1, nwt, vr)):
        def chunk(i, _, idx_hbm=idx_hbm, tab=tab, out=out):
            base = wid * (NP // NWK) + i * 64
            pltpu.sync_copy(idx_hbm.at[pl.ds(base, 64)], idx_v)
            pltpu.async_copy(tab.at[idx_v], rows_v, sem).wait()
            pltpu.sync_copy(rows_v, out.at[pl.ds(base, 64)])
            return 0
        lax.fori_loop(0, NP // NWK // 64, chunk, 0)


def _prep(nf, nwt, ei0, ei1, he1):
    return pl.kernel(
        _prep_body,
        out_type=(_f32(NP, 16), _f32(NP, 16), _f32(NP, 16)),
        mesh=_mesh(),
        compiler_params=pltpu.CompilerParams(use_tc_tiling_on_sc=False),
        scratch_types=[
            pltpu.VMEM((64,), jnp.int32),
            pltpu.VMEM((64, 16), jnp.float32),
            pltpu.SemaphoreType.DMA,
        ],
    )(nf, nwt, ei0, ei1, he1)


def _hop_body(table, idx, zrows, p0, p1, acc, idx_v, rows_v, sem, *,
              cw0, cw1):
    cid = lax.axis_index("c")
    sid = lax.axis_index("s")
    start = jnp.where(cid == 0, sid * cw0, 16 * cw0 + sid * cw1)
    mycw = jnp.where(cid == 0, cw0, cw1)
    rpt = NH // 16
    pltpu.sync_copy(zrows, acc.at[pl.ds(sid * rpt, rpt)])
    plsc.subcore_barrier()

    def visit(t, _):
        pltpu.sync_copy(idx.at[start + t], idx_v)
        pltpu.async_copy(table.at[idx_v.at[0]], rows_v, sem).wait()
        pltpu.sync_copy(rows_v, acc.at[idx_v.at[1]], add=True)
        return 0

    lax.fori_loop(0, mycw, visit, 0)
    plsc.subcore_barrier()

    @pl.when(cid == 0)
    def _():
        pltpu.sync_copy(acc.at[pl.ds(sid * rpt, rpt)],
                        p0.at[pl.ds(sid * rpt, rpt)])

    @pl.when(cid == 1)
    def _():
        pltpu.sync_copy(acc.at[pl.ds(sid * rpt, rpt)],
                        p1.at[pl.ds(sid * rpt, rpt)])


def _hop(table, idx, zrows, width):
    body = functools.partial(_hop_body, cw0=CW0, cw1=CW1)
    return pl.kernel(
        body,
        out_type=(_f32(NH, width), _f32(NH, width)),
        mesh=_mesh(),
        compiler_params=pltpu.CompilerParams(use_tc_tiling_on_sc=False),
        scratch_types=[
            pltpu.VMEM_SHARED((NH, width), jnp.float32),
            pltpu.VMEM((2, CH), jnp.int32),
            pltpu.VMEM((CH, width), jnp.float32),
            pltpu.SemaphoreType.DMA,
        ],
    )(table, idx, zrows)


def _tc(body, out_type, *args):
    return pl.pallas_call(body, out_shape=out_type)(*args)


def _nw_body(x, p, W1, b1, W2, b2, o):
    W = W1[...]
    h = (lax.dot_general(x[...], W[:, :64], (((1,), (1,)), ((), ())))
         + lax.dot_general(p[...], W[:, 64:], (((1,), (1,)), ((), ())))
         + b1[...])
    h = jax.nn.relu(h)
    W2r = jnp.broadcast_to(W2[...], (16, 64))
    nw16 = jax.nn.sigmoid(
        lax.dot_general(h, W2r, (((1,), (1,)), ((), ()))) + b2[...][0, 0])
    o[...] = jnp.concatenate(
        [nw16, jnp.zeros((NP - N, 16), jnp.float32)], axis=0)


def _t1_body(er, fu, fv, Wc1, o):
    ef = (fu[...][:N] + fv[...][:N]) * 0.5
    W = Wc1[...]
    xx = (lax.dot_general(er[...], W[:, :112], (((1,), (1,)), ((), ())))
          + lax.dot_general(ef, W[:, 112:], (((1,), (1,)), ((), ()))))
    xxp = jnp.concatenate([xx, jnp.zeros((NH - N, F1), jnp.float32)], axis=0)
    lane = lax.broadcasted_iota(jnp.int32, (NH, 16), 1)
    ones_slab = jnp.where(lane == 0, 1.0, 0.0).astype(jnp.float32)
    o[...] = jnp.concatenate([xxp, ones_slab], axis=1)


def _bcast(col, k):
    return lax.dot_general(col, jnp.ones((1, k), jnp.float32),
                           (((1,), (0,)), ((), ())))


def _t2_body(p0, p1, v16, o_tab, o_binv):
    s = p0[...] + p1[...]
    b = s[:, 128:129]
    binv = jnp.where(b == 0, 0.0, 1.0 / jnp.where(b == 0, 1.0, b))
    t = s[:, :F1] * _bcast(binv, F1)
    lane = lax.broadcasted_iota(jnp.int32, (NH, 16), 1)
    vslab = jnp.where(lane == 0, v16[...][:NH], 0.0)
    o_tab[...] = jnp.concatenate([t, vslab], axis=1)
    o_binv[...] = _bcast(binv, F2)


def _t3_body(p0, p1, bc1, Wc2, o_tab, o_dinv):
    s = p0[...] + p1[...]
    d = s[:, 128:129]
    dinv = jnp.where(d == 0, 0.0, 1.0 / jnp.where(d == 0, 1.0, d))
    er1 = jax.nn.sigmoid(s[:, :F1] * _bcast(dinv, F1) + bc1[...])
    o_tab[...] = lax.dot_general(er1, Wc2[...], (((1,), (1,)), ((), ())))
    o_dinv[...] = _bcast(dinv, F2)


def _t4_body(p0, p1, binv, o):
    o[...] = (p0[...] + p1[...]) * binv[...]


def _fin_body(p0, p1, dinv, bc2, o):
    i = pl.program_id(0)
    bc = bc2[...]

    @pl.when(i < 10)
    def _():
        o[...] = jax.nn.sigmoid((p0[...] + p1[...]) * dinv[...] + bc)

    @pl.when(i >= 10)
    def _():
        o[...] = jnp.broadcast_to(jax.nn.sigmoid(bc), (1000, F2))


def kernel(edge_index, edge_rep, x, hyper_edge, prototype, node_feature,
           W1, b1, W2, b2, Wc1, bc1, Wc2, bc2):
    ei = edge_index.astype(jnp.int32)
    he = hyper_edge.astype(jnp.int32)

    def padn(a):
        return jnp.pad(a, (0, NP - N))

    ei0 = padn(ei[0, :N])
    ei1 = padn(ei[1, :N])
    he1t = padn(he[1, :N])
    spare = N + jnp.arange(MP - M, dtype=jnp.int32) % (NH - N)

    def pack(src, dst):
        s3 = jnp.pad(src, (0, MP - M)).reshape(MP // CH, 1, CH)
        d3 = jnp.concatenate([dst, spare]).reshape(MP // CH, 1, CH)
        return jnp.concatenate([s3, d3], axis=1)

    idx_a = pack(he[0], he[1])
    idx_b = pack(he[1], he[0])

    z1 = jnp.zeros((NH // 16, F1P), jnp.float32)
    z2 = jnp.zeros((NH // 16, F2), jnp.float32)

    nwt = _tc(_nw_body, _f32(NP, 16),
              x, prototype, W1, b1.reshape(1, 64), W2, b2.reshape(1, 1))

    fu, fv, vr = _prep(node_feature, nwt, ei0, ei1, he1t)

    tab1 = _tc(_t1_body, _f32(NH, F1P), edge_rep[:N], fu, fv, Wc1)

    p0, p1 = _hop(tab1, idx_a, z1, F1P)
    tab2, binv = _tc(_t2_body, (_f32(NH, F1P), _f32(NH, F2)), p0, p1, vr)
    p0, p1 = _hop(tab2, idx_b, z1, F1P)
    tab3, dinv = _tc(_t3_body, (_f32(NH, F2), _f32(NH, F2)),
                     p0, p1, bc1.reshape(1, F1), Wc2)
    p0, p1 = _hop(tab3, idx_a, z2, F2)
    tab4 = _tc(_t4_body, _f32(NH, F2), p0, p1, binv)
    p0, p1 = _hop(tab4, idx_b, z2, F2)

    capped = lambda i: (jnp.minimum(i, 9), 0)
    out = pl.pallas_call(
        _fin_body,
        out_shape=_f32(E, F2),
        grid=(E // 1000,),
        in_specs=[
            pl.BlockSpec((1000, F2), capped),
            pl.BlockSpec((1000, F2), capped),
            pl.BlockSpec((1000, F2), capped),
            pl.BlockSpec((1, F2), lambda i: (0, 0)),
        ],
        out_specs=pl.BlockSpec((1000, F2), lambda i: (i, 0)),
    )(p0, p1, dinv, bc2.reshape(1, F2))
    return out


# speedup vs baseline: 1.4523x; 1.4523x over previous
import functools

import jax
import jax.numpy as jnp
from jax import lax
from jax.experimental import pallas as pl
from jax.experimental.pallas import tpu as pltpu
from jax.experimental.pallas import tpu_sc as plsc

N = 10000
E = 320000
M = 640000
NP = 10240
MP = 643072
F1 = 128
F1P = 144
F2 = 64
CH = 128
NWK = 32
DUMMY = 10016

_mesh = functools.partial(
    plsc.VectorSubcoreMesh, core_axis_name="c", subcore_axis_name="s")


def _f32(*shape):
    return jax.ShapeDtypeStruct(shape, jnp.float32)


def _prep_body(nf, nwt, ei0, ei1, he1, fu, fv, vr, idx_v, rows_v, sem):
    cid = lax.axis_index("c")
    sid = lax.axis_index("s")
    wid = sid * 2 + cid
    for idx_hbm, tab, out in ((ei0, nf, fu), (ei1, nf, fv), (he1, nwt, vr)):
        def chunk(i, _, idx_hbm=idx_hbm, tab=tab, out=out):
            base = wid * (NP // NWK) + i * 64
            pltpu.sync_copy(idx_hbm.at[pl.ds(base, 64)], idx_v)
            pltpu.async_copy(tab.at[idx_v], rows_v, sem).wait()
            pltpu.sync_copy(rows_v, out.at[pl.ds(base, 64)])
            return 0
        lax.fori_loop(0, NP // NWK // 64, chunk, 0)


def _prep(nf, nwt, ei0, ei1, he1):
    return pl.kernel(
        _prep_body,
        out_type=(_f32(NP, 16), _f32(NP, 16), _f32(NP, 16)),
        mesh=_mesh(),
        compiler_params=pltpu.CompilerParams(use_tc_tiling_on_sc=False),
        scratch_types=[
            pltpu.VMEM((64,), jnp.int32),
            pltpu.VMEM((64, 16), jnp.float32),
            pltpu.SemaphoreType.DMA,
        ],
    )(nf, nwt, ei0, ei1, he1)


def _hop_body(table, sidx, didx, zrows, p0, p1,
              acc, sidx_v, didx_v, rows_v, sem):
    cid = lax.axis_index("c")
    sid = lax.axis_index("s")
    wid = sid * 2 + cid
    rows_per_tile = NP // 16
    pltpu.sync_copy(zrows, acc.at[pl.ds(sid * rows_per_tile, rows_per_tile)])
    plsc.subcore_barrier()

    def chunk(i, _):
        base = wid * (MP // NWK) + i * CH
        pltpu.sync_copy(sidx.at[pl.ds(base, CH)], sidx_v)
        pltpu.async_copy(table.at[sidx_v], rows_v, sem).wait()
        pltpu.sync_copy(didx.at[pl.ds(base, CH)], didx_v)
        pltpu.sync_copy(rows_v, acc.at[didx_v], add=True)
        return 0

    lax.fori_loop(0, MP // NWK // CH, chunk, 0)
    plsc.subcore_barrier()

    @pl.when(cid == 0)
    def _():
        pltpu.sync_copy(acc.at[pl.ds(sid * rows_per_tile, rows_per_tile)],
                        p0.at[pl.ds(sid * rows_per_tile, rows_per_tile)])

    @pl.when(cid == 1)
    def _():
        pltpu.sync_copy(acc.at[pl.ds(sid * rows_per_tile, rows_per_tile)],
                        p1.at[pl.ds(sid * rows_per_tile, rows_per_tile)])


def _hop(table, sidx, didx, zrows, width):
    return pl.kernel(
        _hop_body,
        out_type=(_f32(NP, width), _f32(NP, width)),
        mesh=_mesh(),
        compiler_params=pltpu.CompilerParams(use_tc_tiling_on_sc=False),
        scratch_types=[
            pltpu.VMEM_SHARED((NP, width), jnp.float32),
            pltpu.VMEM((CH,), jnp.int32),
            pltpu.VMEM((CH,), jnp.int32),
            pltpu.VMEM((CH, width), jnp.float32),
            pltpu.SemaphoreType.DMA,
        ],
    )(table, sidx, didx, zrows)


def _tc(body, out_type, *args):
    return pl.pallas_call(body, out_shape=out_type)(*args)


def _nw_body(x, p, W1, b1, W2, b2, o):
    W = W1[...]
    h = (lax.dot_general(x[...], W[:, :64], (((1,), (1,)), ((), ())))
         + lax.dot_general(p[...], W[:, 64:], (((1,), (1,)), ((), ())))
         + b1[...])
    h = jax.nn.relu(h)
    W2r = jnp.broadcast_to(W2[...], (16, 64))
    nw16 = jax.nn.sigmoid(
        lax.dot_general(h, W2r, (((1,), (1,)), ((), ()))) + b2[...][0, 0])
    o[...] = jnp.concatenate(
        [nw16, jnp.zeros((NP - N, 16), jnp.float32)], axis=0)


def _t1_body(er, fu, fv, Wc1, o):
    ef = (fu[...][:N] + fv[...][:N]) * 0.5
    W = Wc1[...]
    xx = (lax.dot_general(er[...], W[:, :112], (((1,), (1,)), ((), ())))
          + lax.dot_general(ef, W[:, 112:], (((1,), (1,)), ((), ()))))
    xxp = jnp.concatenate([xx, jnp.zeros((NP - N, F1), jnp.float32)], axis=0)
    lane = lax.broadcasted_iota(jnp.int32, (NP, 16), 1)
    ones_slab = jnp.where(lane == 0, 1.0, 0.0).astype(jnp.float32)
    o[...] = jnp.concatenate([xxp, ones_slab], axis=1)


def _bcast(col, k):
    return lax.dot_general(col, jnp.ones((1, k), jnp.float32),
                           (((1,), (0,)), ((), ())))


def _t2_body(p0, p1, v16, o_tab, o_binv):
    s = p0[...] + p1[...]
    b = s[:, 128:129]
    binv = jnp.where(b == 0, 0.0, 1.0 / jnp.where(b == 0, 1.0, b))
    t = s[:, :F1] * _bcast(binv, F1)
    lane = lax.broadcasted_iota(jnp.int32, (NP, 16), 1)
    vslab = jnp.where(lane == 0, v16[...], 0.0)
    o_tab[...] = jnp.concatenate([t, vslab], axis=1)
    o_binv[...] = _bcast(binv, F2)


def _t3_body(p0, p1, bc1, Wc2, o_tab, o_dinv):
    s = p0[...] + p1[...]
    d = s[:, 128:129]
    dinv = jnp.where(d == 0, 0.0, 1.0 / jnp.where(d == 0, 1.0, d))
    er1 = jax.nn.sigmoid(s[:, :F1] * _bcast(dinv, F1) + bc1[...])
    o_tab[...] = lax.dot_general(er1, Wc2[...], (((1,), (1,)), ((), ())))
    o_dinv[...] = _bcast(dinv, F2)


def _t4_body(p0, p1, binv, o):
    o[...] = (p0[...] + p1[...]) * binv[...]


def _fin_body(p0, p1, dinv, bc2, o):
    i = pl.program_id(0)
    bc = bc2[...]

    @pl.when(i < 10)
    def _():
        o[...] = jax.nn.sigmoid((p0[...] + p1[...]) * dinv[...] + bc)

    @pl.when(i >= 10)
    def _():
        o[...] = jnp.broadcast_to(jax.nn.sigmoid(bc), (1000, F2))


def kernel(edge_index, edge_rep, x, hyper_edge, prototype, node_feature,
           W1, b1, W2, b2, Wc1, bc1, Wc2, bc2):
    ei = edge_index.astype(jnp.int32)
    he = hyper_edge.astype(jnp.int32)

    def padn(a):
        return jnp.pad(a, (0, NP - N))

    ei0 = padn(ei[0, :N])
    ei1 = padn(ei[1, :N])
    he1t = padn(he[1, :N])
    h0s = jnp.pad(he[0], (0, MP - M))
    h1s = jnp.pad(he[1], (0, MP - M))
    h0d = jnp.pad(he[0], (0, MP - M), constant_values=DUMMY)
    h1d = jnp.pad(he[1], (0, MP - M), constant_values=DUMMY)

    z1 = jnp.zeros((NP // 16, F1P), jnp.float32)
    z2 = jnp.zeros((NP // 16, F2), jnp.float32)

    nwt = _tc(_nw_body, _f32(NP, 16),
              x, prototype, W1, b1.reshape(1, 64), W2, b2.reshape(1, 1))

    fu, fv, vr = _prep(node_feature, nwt, ei0, ei1, he1t)

    tab1 = _tc(_t1_body, _f32(NP, F1P), edge_rep[:N], fu, fv, Wc1)

    p0, p1 = _hop(tab1, h0s, h1d, z1, F1P)
    tab2, binv = _tc(_t2_body, (_f32(NP, F1P), _f32(NP, F2)), p0, p1, vr)
    p0, p1 = _hop(tab2, h1s, h0d, z1, F1P)
    tab3, dinv = _tc(_t3_body, (_f32(NP, F2), _f32(NP, F2)),
                     p0, p1, bc1.reshape(1, F1), Wc2)
    p0, p1 = _hop(tab3, h0s, h1d, z2, F2)
    tab4 = _tc(_t4_body, _f32(NP, F2), p0, p1, binv)
    p0, p1 = _hop(tab4, h1s, h0d, z2, F2)

    capped = lambda i: (jnp.minimum(i, 9), 0)
    out = pl.pallas_call(
        _fin_body,
        out_shape=_f32(E, F2),
        grid=(E // 1000,),
        in_specs=[
            pl.BlockSpec((1000, F2), capped),
            pl.BlockSpec((1000, F2), capped),
            pl.BlockSpec((1000, F2), capped),
            pl.BlockSpec((1, F2), lambda i: (0, 0)),
        ],
        out_specs=pl.BlockSpec((1000, F2), lambda i: (i, 0)),
    )(p0, p1, dinv, bc2.reshape(1, F2))
    return out


# speedup vs baseline: 1.5108x; 1.0403x over previous
import functools

import jax
import jax.numpy as jnp
from jax import lax
from jax.experimental import pallas as pl
from jax.experimental.pallas import tpu as pltpu
from jax.experimental.pallas import tpu_sc as plsc

N = 10000
E = 320000
M = 640000
NP = 10240
MP = 643072
F1 = 128
F1P = 144
F2 = 64
CH = 128
NWK = 32
DUMMY = 10016

_mesh = functools.partial(
    plsc.VectorSubcoreMesh, core_axis_name="c", subcore_axis_name="s")


def _f32(*shape):
    return jax.ShapeDtypeStruct(shape, jnp.float32)


def _prep_body(nf, nwt, ei0, ei1, he1, fu, fv, vr, idx_v, rows_v, sem):
    cid = lax.axis_index("c")
    sid = lax.axis_index("s")
    wid = sid * 2 + cid
    for idx_hbm, tab, out in ((ei0, nf, fu), (ei1, nf, fv), (he1, nwt, vr)):
        def chunk(i, _, idx_hbm=idx_hbm, tab=tab, out=out):
            base = wid * (NP // NWK) + i * 64
            pltpu.sync_copy(idx_hbm.at[pl.ds(base, 64)], idx_v)
            pltpu.async_copy(tab.at[idx_v], rows_v, sem).wait()
            pltpu.sync_copy(rows_v, out.at[pl.ds(base, 64)])
            return 0
        lax.fori_loop(0, NP // NWK // 64, chunk, 0)


def _prep(nf, nwt, ei0, ei1, he1):
    return pl.kernel(
        _prep_body,
        out_type=(_f32(NP, 16), _f32(NP, 16), _f32(NP, 16)),
        mesh=_mesh(),
        compiler_params=pltpu.CompilerParams(use_tc_tiling_on_sc=False),
        scratch_types=[
            pltpu.VMEM((64,), jnp.int32),
            pltpu.VMEM((64, 16), jnp.float32),
            pltpu.SemaphoreType.DMA,
        ],
    )(nf, nwt, ei0, ei1, he1)


def _hop_body(table, sidx, didx, zrows, p0, p1,
              acc, sidx_v, didx_v, rows_v, sem):
    cid = lax.axis_index("c")
    sid = lax.axis_index("s")
    wid = sid * 2 + cid
    rows_per_tile = NP // 16
    pltpu.sync_copy(zrows, acc.at[pl.ds(sid * rows_per_tile, rows_per_tile)])
    plsc.subcore_barrier()

    def chunk(i, _):
        base = wid * (MP // NWK) + i * CH
        pltpu.sync_copy(sidx.at[pl.ds(base, CH)], sidx_v)
        pltpu.async_copy(table.at[sidx_v], rows_v, sem).wait()
        pltpu.sync_copy(didx.at[pl.ds(base, CH)], didx_v)
        pltpu.sync_copy(rows_v, acc.at[didx_v], add=True)
        return 0

    lax.fori_loop(0, MP // NWK // CH, chunk, 0)
    plsc.subcore_barrier()

    @pl.when(cid == 0)
    def _():
        pltpu.sync_copy(acc.at[pl.ds(sid * rows_per_tile, rows_per_tile)],
                        p0.at[pl.ds(sid * rows_per_tile, rows_per_tile)])

    @pl.when(cid == 1)
    def _():
        pltpu.sync_copy(acc.at[pl.ds(sid * rows_per_tile, rows_per_tile)],
                        p1.at[pl.ds(sid * rows_per_tile, rows_per_tile)])


def _hop(table, sidx, didx, zrows, width):
    return pl.kernel(
        _hop_body,
        out_type=(_f32(NP, width), _f32(NP, width)),
        mesh=_mesh(),
        compiler_params=pltpu.CompilerParams(use_tc_tiling_on_sc=False),
        scratch_types=[
            pltpu.VMEM_SHARED((NP, width), jnp.float32),
            pltpu.VMEM((CH,), jnp.int32),
            pltpu.VMEM((CH,), jnp.int32),
            pltpu.VMEM((CH, width), jnp.float32),
            pltpu.SemaphoreType.DMA,
        ],
    )(table, sidx, didx, zrows)


def _tc(body, out_type, *args):
    return pl.pallas_call(body, out_shape=out_type)(*args)


def _nw_body(x, p, W1, b1, W2, b2, o):
    W = W1[...]
    h = (lax.dot_general(x[...], W[:, :64], (((1,), (1,)), ((), ())))
         + lax.dot_general(p[...], W[:, 64:], (((1,), (1,)), ((), ())))
         + b1[...])
    h = jax.nn.relu(h)
    W2r = jnp.broadcast_to(W2[...], (16, 64))
    nw16 = jax.nn.sigmoid(
        lax.dot_general(h, W2r, (((1,), (1,)), ((), ()))) + b2[...][0, 0])
    o[...] = jnp.concatenate(
        [nw16, jnp.zeros((NP - N, 16), jnp.float32)], axis=0)


def _t1_body(er, fu, fv, Wc1, o):
    ef = (fu[...][:N] + fv[...][:N]) * 0.5
    W = Wc1[...]
    xx = (lax.dot_general(er[...], W[:, :112], (((1,), (1,)), ((), ())))
          + lax.dot_general(ef, W[:, 112:], (((1,), (1,)), ((), ()))))
    xxp = jnp.concatenate([xx, jnp.zeros((NP - N, F1), jnp.float32)], axis=0)
    lane = lax.broadcasted_iota(jnp.int32, (NP, 16), 1)
    ones_slab = jnp.where(lane == 0, 1.0, 0.0).astype(jnp.float32)
    o[...] = jnp.concatenate([xxp, ones_slab], axis=1)


def _bcast(col, k):
    return lax.dot_general(col, jnp.ones((1, k), jnp.float32),
                           (((1,), (0,)), ((), ())))


def _t2_body(p0, p1, v16, o_tab, o_binv):
    s = p0[...] + p1[...]
    b = s[:, 128:129]
    binv = jnp.where(b == 0, 0.0, 1.0 / jnp.where(b == 0, 1.0, b))
    t = s[:, :F1] * _bcast(binv, F1)
    lane = lax.broadcasted_iota(jnp.int32, (NP, 16), 1)
    vslab = jnp.where(lane == 0, v16[...], 0.0)
    o_tab[...] = jnp.concatenate([t, vslab], axis=1)
    o_binv[...] = _bcast(binv, F2)


def _t3_body(p0, p1, bc1, Wc2, o_tab, o_dinv):
    s = p0[...] + p1[...]
    d = s[:, 128:129]
    dinv = jnp.where(d == 0, 0.0, 1.0 / jnp.where(d == 0, 1.0, d))
    er1 = jax.nn.sigmoid(s[:, :F1] * _bcast(dinv, F1) + bc1[...])
    o_tab[...] = lax.dot_general(er1, Wc2[...], (((1,), (1,)), ((), ())))
    o_dinv[...] = _bcast(dinv, F2)


def _t4_body(p0, p1, binv, o):
    o[...] = (p0[...] + p1[...]) * binv[...]


def _prefill_body(bc2, o):
    o[...] = jnp.broadcast_to(jax.nn.sigmoid(bc2[...]), (1000, F2))


def _fin_body(pre, p0, p1, dinv, bc2, o):
    o[...] = jax.nn.sigmoid((p0[...] + p1[...]) * dinv[...] + bc2[...])


def kernel(edge_index, edge_rep, x, hyper_edge, prototype, node_feature,
           W1, b1, W2, b2, Wc1, bc1, Wc2, bc2):
    ei = edge_index.astype(jnp.int32)
    he = hyper_edge.astype(jnp.int32)

    def padn(a):
        return jnp.pad(a, (0, NP - N))

    ei0 = padn(ei[0, :N])
    ei1 = padn(ei[1, :N])
    he1t = padn(he[1, :N])
    h0s = jnp.pad(he[0], (0, MP - M))
    h1s = jnp.pad(he[1], (0, MP - M))
    h0d = jnp.pad(he[0], (0, MP - M), constant_values=DUMMY)
    h1d = jnp.pad(he[1], (0, MP - M), constant_values=DUMMY)

    z1 = jnp.zeros((NP // 16, F1P), jnp.float32)
    z2 = jnp.zeros((NP // 16, F2), jnp.float32)

    nwt = _tc(_nw_body, _f32(NP, 16),
              x, prototype, W1, b1.reshape(1, 64), W2, b2.reshape(1, 1))

    fu, fv, vr = _prep(node_feature, nwt, ei0, ei1, he1t)

    tab1 = _tc(_t1_body, _f32(NP, F1P), edge_rep[:N], fu, fv, Wc1)

    p0, p1 = _hop(tab1, h0s, h1d, z1, F1P)
    tab2, binv = _tc(_t2_body, (_f32(NP, F1P), _f32(NP, F2)), p0, p1, vr)
    p0, p1 = _hop(tab2, h1s, h0d, z1, F1P)
    tab3, dinv = _tc(_t3_body, (_f32(NP, F2), _f32(NP, F2)),
                     p0, p1, bc1.reshape(1, F1), Wc2)
    p0, p1 = _hop(tab3, h0s, h1d, z2, F2)
    tab4 = _tc(_t4_body, _f32(NP, F2), p0, p1, binv)
    p0, p1 = _hop(tab4, h1s, h0d, z2, F2)

    bc2r = bc2.reshape(1, F2)
    pre = pl.pallas_call(
        _prefill_body,
        out_shape=_f32(E, F2),
        grid=(E // 1000,),
        in_specs=[pl.BlockSpec((1, F2), lambda i: (0, 0))],
        out_specs=pl.BlockSpec((1000, F2), lambda i: (i, 0)),
    )(bc2r)
    out = pl.pallas_call(
        _fin_body,
        out_shape=_f32(E, F2),
        grid=(N // 1000,),
        in_specs=[
            pl.BlockSpec(memory_space=pl.MemorySpace.ANY),
            pl.BlockSpec((1000, F2), lambda i: (i, 0)),
            pl.BlockSpec((1000, F2), lambda i: (i, 0)),
            pl.BlockSpec((1000, F2), lambda i: (i, 0)),
            pl.BlockSpec((1, F2), lambda i: (0, 0)),
        ],
        out_specs=pl.BlockSpec((1000, F2), lambda i: (i, 0)),
        input_output_aliases={0: 0},
    )(pre, p0, p1, dinv, bc2r)
    return out


# speedup vs baseline: 1.7310x; 1.1457x over previous
import functools

import jax
import jax.numpy as jnp
from jax import lax
from jax.experimental import pallas as pl
from jax.experimental.pallas import tpu as pltpu
from jax.experimental.pallas import tpu_sc as plsc

N = 10000
E = 320000
M = 640000
NP = 10240
MP = 643072
F1 = 128
F1P = 144
F2 = 64
CH = 128
NWK = 32
DUMMY = 10016

_mesh = functools.partial(
    plsc.VectorSubcoreMesh, core_axis_name="c", subcore_axis_name="s")


def _f32(*shape):
    return jax.ShapeDtypeStruct(shape, jnp.float32)


def _prep_body(nf, nwt, ei0, ei1, he1, fu, fv, vr, idx_v, rows_v, sem):
    cid = lax.axis_index("c")
    sid = lax.axis_index("s")
    wid = sid * 2 + cid
    for idx_hbm, tab, out in ((ei0, nf, fu), (ei1, nf, fv), (he1, nwt, vr)):
        def chunk(i, _, idx_hbm=idx_hbm, tab=tab, out=out):
            base = wid * (NP // NWK) + i * 64
            pltpu.sync_copy(idx_hbm.at[pl.ds(base, 64)], idx_v)
            pltpu.async_copy(tab.at[idx_v], rows_v, sem).wait()
            pltpu.sync_copy(rows_v, out.at[pl.ds(base, 64)])
            return 0
        lax.fori_loop(0, NP // NWK // 64, chunk, 0)


def _prep(nf, nwt, ei0, ei1, he1):
    return pl.kernel(
        _prep_body,
        out_type=(_f32(NP, 16), _f32(NP, 16), _f32(NP, 16)),
        mesh=_mesh(),
        compiler_params=pltpu.CompilerParams(use_tc_tiling_on_sc=False),
        scratch_types=[
            pltpu.VMEM((64,), jnp.int32),
            pltpu.VMEM((64, 16), jnp.float32),
            pltpu.SemaphoreType.DMA,
        ],
    )(nf, nwt, ei0, ei1, he1)


def _hop_body(table, idx, zrows, p0, p1, acc, idx_v, rows_v, sem):
    cid = lax.axis_index("c")
    sid = lax.axis_index("s")
    wid = sid * 2 + cid
    rows_per_tile = NP // 16
    pltpu.sync_copy(zrows, acc.at[pl.ds(sid * rows_per_tile, rows_per_tile)])
    plsc.subcore_barrier()

    def chunk(i, _):
        c = wid * (MP // NWK // CH) + i
        pltpu.sync_copy(idx.at[c], idx_v)
        pltpu.async_copy(table.at[idx_v.at[0]], rows_v, sem).wait()
        pltpu.sync_copy(rows_v, acc.at[idx_v.at[1]], add=True)
        return 0

    lax.fori_loop(0, MP // NWK // CH, chunk, 0)
    plsc.subcore_barrier()

    @pl.when(cid == 0)
    def _():
        pltpu.sync_copy(acc.at[pl.ds(sid * rows_per_tile, rows_per_tile)],
                        p0.at[pl.ds(sid * rows_per_tile, rows_per_tile)])

    @pl.when(cid == 1)
    def _():
        pltpu.sync_copy(acc.at[pl.ds(sid * rows_per_tile, rows_per_tile)],
                        p1.at[pl.ds(sid * rows_per_tile, rows_per_tile)])


def _hop(table, idx, zrows, width):
    return pl.kernel(
        _hop_body,
        out_type=(_f32(NP, width), _f32(NP, width)),
        mesh=_mesh(),
        compiler_params=pltpu.CompilerParams(use_tc_tiling_on_sc=False),
        scratch_types=[
            pltpu.VMEM_SHARED((NP, width), jnp.float32),
            pltpu.VMEM((2, CH), jnp.int32),
            pltpu.VMEM((CH, width), jnp.float32),
            pltpu.SemaphoreType.DMA,
        ],
    )(table, idx, zrows)


def _tc(body, out_type, *args):
    return pl.pallas_call(body, out_shape=out_type)(*args)


def _nw_body(x, p, W1, b1, W2, b2, o):
    W = W1[...]
    h = (lax.dot_general(x[...], W[:, :64], (((1,), (1,)), ((), ())))
         + lax.dot_general(p[...], W[:, 64:], (((1,), (1,)), ((), ())))
         + b1[...])
    h = jax.nn.relu(h)
    W2r = jnp.broadcast_to(W2[...], (16, 64))
    nw16 = jax.nn.sigmoid(
        lax.dot_general(h, W2r, (((1,), (1,)), ((), ()))) + b2[...][0, 0])
    o[...] = jnp.concatenate(
        [nw16, jnp.zeros((NP - N, 16), jnp.float32)], axis=0)


def _t1_body(er, fu, fv, Wc1, o):
    ef = (fu[...][:N] + fv[...][:N]) * 0.5
    W = Wc1[...]
    xx = (lax.dot_general(er[...], W[:, :112], (((1,), (1,)), ((), ())))
          + lax.dot_general(ef, W[:, 112:], (((1,), (1,)), ((), ()))))
    xxp = jnp.concatenate([xx, jnp.zeros((NP - N, F1), jnp.float32)], axis=0)
    lane = lax.broadcasted_iota(jnp.int32, (NP, 16), 1)
    ones_slab = jnp.where(lane == 0, 1.0, 0.0).astype(jnp.float32)
    o[...] = jnp.concatenate([xxp, ones_slab], axis=1)


def _bcast(col, k):
    return lax.dot_general(col, jnp.ones((1, k), jnp.float32),
                           (((1,), (0,)), ((), ())))


def _t2_body(p0, p1, v16, o_tab, o_binv):
    s = p0[...] + p1[...]
    b = s[:, 128:129]
    binv = jnp.where(b == 0, 0.0, 1.0 / jnp.where(b == 0, 1.0, b))
    t = s[:, :F1] * _bcast(binv, F1)
    lane = lax.broadcasted_iota(jnp.int32, (NP, 16), 1)
    vslab = jnp.where(lane == 0, v16[...], 0.0)
    o_tab[...] = jnp.concatenate([t, vslab], axis=1)
    o_binv[...] = _bcast(binv, F2)


def _t3_body(p0, p1, bc1, Wc2, o_tab, o_dinv):
    s = p0[...] + p1[...]
    d = s[:, 128:129]
    dinv = jnp.where(d == 0, 0.0, 1.0 / jnp.where(d == 0, 1.0, d))
    er1 = jax.nn.sigmoid(s[:, :F1] * _bcast(dinv, F1) + bc1[...])
    o_tab[...] = lax.dot_general(er1, Wc2[...], (((1,), (1,)), ((), ())))
    o_dinv[...] = _bcast(dinv, F2)


def _t4_body(p0, p1, binv, o):
    o[...] = (p0[...] + p1[...]) * binv[...]


def _prefill_body(bc2, o):
    o[...] = jnp.broadcast_to(jax.nn.sigmoid(bc2[...]), (1000, F2))


def _fin_body(pre, p0, p1, dinv, bc2, o):
    o[...] = jax.nn.sigmoid((p0[...] + p1[...]) * dinv[...] + bc2[...])


def kernel(edge_index, edge_rep, x, hyper_edge, prototype, node_feature,
           W1, b1, W2, b2, Wc1, bc1, Wc2, bc2):
    ei = edge_index.astype(jnp.int32)
    he = hyper_edge.astype(jnp.int32)

    def padn(a):
        return jnp.pad(a, (0, NP - N))

    ei0 = padn(ei[0, :N])
    ei1 = padn(ei[1, :N])
    he1t = padn(he[1, :N])
    def pack(src, dst):
        s3 = jnp.pad(src, (0, MP - M)).reshape(MP // CH, 1, CH)
        d3 = jnp.pad(dst, (0, MP - M),
                     constant_values=DUMMY).reshape(MP // CH, 1, CH)
        return jnp.concatenate([s3, d3], axis=1)

    idx_a = pack(he[0], he[1])
    idx_b = pack(he[1], he[0])

    z1 = jnp.zeros((NP // 16, F1P), jnp.float32)
    z2 = jnp.zeros((NP // 16, F2), jnp.float32)

    nwt = _tc(_nw_body, _f32(NP, 16),
              x, prototype, W1, b1.reshape(1, 64), W2, b2.reshape(1, 1))

    fu, fv, vr = _prep(node_feature, nwt, ei0, ei1, he1t)

    tab1 = _tc(_t1_body, _f32(NP, F1P), edge_rep[:N], fu, fv, Wc1)

    p0, p1 = _hop(tab1, idx_a, z1, F1P)
    tab2, binv = _tc(_t2_body, (_f32(NP, F1P), _f32(NP, F2)), p0, p1, vr)
    p0, p1 = _hop(tab2, idx_b, z1, F1P)
    tab3, dinv = _tc(_t3_body, (_f32(NP, F2), _f32(NP, F2)),
                     p0, p1, bc1.reshape(1, F1), Wc2)
    p0, p1 = _hop(tab3, idx_a, z2, F2)
    tab4 = _tc(_t4_body, _f32(NP, F2), p0, p1, binv)
    p0, p1 = _hop(tab4, idx_b, z2, F2)

    bc2r = bc2.reshape(1, F2)
    pre = pl.pallas_call(
        _prefill_body,
        out_shape=_f32(E, F2),
        grid=(E // 1000,),
        in_specs=[pl.BlockSpec((1, F2), lambda i: (0, 0))],
        out_specs=pl.BlockSpec((1000, F2), lambda i: (i, 0)),
    )(bc2r)
    out = pl.pallas_call(
        _fin_body,
        out_shape=_f32(E, F2),
        grid=(N // 1000,),
        in_specs=[
            pl.BlockSpec(memory_space=pl.MemorySpace.ANY),
            pl.BlockSpec((1000, F2), lambda i: (i, 0)),
            pl.BlockSpec((1000, F2), lambda i: (i, 0)),
            pl.BlockSpec((1000, F2), lambda i: (i, 0)),
            pl.BlockSpec((1, F2), lambda i: (0, 0)),
        ],
        out_specs=pl.BlockSpec((1000, F2), lambda i: (i, 0)),
        input_output_aliases={0: 0},
    )(pre, p0, p1, dinv, bc2r)
    return out
